# baseline (speedup 1.0000x reference)
import functools

import jax
import jax.numpy as jnp
from jax import lax
from jax.experimental import pallas as pl
from jax.experimental.pallas import tpu as pltpu
from jax.experimental.pallas import tpu_sc as plsc

VOCAB = 100000
EMBED_DIM = 64
BATCH = 4096
HIST_LEN = 200

N = BATCH * HIST_LEN
NC = 2
NS = 16
NW = NC * NS
B_PER_W = N // NW
CHUNK = 512
NBUF = 2
N_CHUNKS = B_PER_W // CHUNK
N_STEPS = N_CHUNKS // NBUF


def _gather_body(table_hbm, idx_hbm, out_hbm, idx_v, rows_v, *sems):
    gsems = sems[:NBUF]
    ssems = sems[NBUF:]
    wid = lax.axis_index("s") * NC + lax.axis_index("c")
    base = wid * B_PER_W
    pltpu.sync_copy(idx_hbm.at[pl.ds(base, B_PER_W)], idx_v)

    def rows_ref(b):
        return rows_v.at[pl.ds(b * CHUNK, CHUNK)]

    def fire_gather(c, b):
        pltpu.async_copy(
            table_hbm.at[idx_v.at[pl.ds(c * CHUNK, CHUNK)]], rows_ref(b), gsems[b]
        )

    def wait_gather(b):
        pltpu.make_async_copy(
            table_hbm.at[idx_v.at[pl.ds(0, CHUNK)]], rows_ref(b), gsems[b]
        ).wait()

    def fire_store(c, b):
        pltpu.async_copy(
            rows_ref(b), out_hbm.at[pl.ds(base + c * CHUNK, CHUNK)], ssems[b]
        )

    def wait_store(b):
        pltpu.make_async_copy(
            rows_ref(b), out_hbm.at[pl.ds(base, CHUNK)], ssems[b]
        ).wait()

    for b in range(NBUF):
        fire_gather(b, b)

    @pl.loop(0, N_STEPS - 1)
    def _step(s):
        for b in range(NBUF):
            wait_gather(b)
            fire_store(s * NBUF + b, b)
        for b in range(NBUF):
            wait_store(b)
            fire_gather(s * NBUF + b + NBUF, b)

    for b in range(NBUF):
        c = (N_STEPS - 1) * NBUF + b
        wait_gather(b)
        fire_store(c, b)
    for b in range(NBUF):
        wait_store(b)


@jax.jit
def _gather(table, idx_flat):
    run = pl.kernel(
        _gather_body,
        out_type=jax.ShapeDtypeStruct((N, EMBED_DIM), jnp.float32),
        mesh=plsc.VectorSubcoreMesh(core_axis_name="c", subcore_axis_name="s"),
        scratch_types=[
            pltpu.VMEM((B_PER_W,), jnp.int32),
            pltpu.VMEM((NBUF * CHUNK, EMBED_DIM), jnp.float32),
        ]
        + [pltpu.SemaphoreType.DMA] * (2 * NBUF),
        compiler_params=pltpu.CompilerParams(use_tc_tiling_on_sc=False),
    )
    return run(table, idx_flat)


def kernel(table, index):
    idx_flat = index.reshape(-1).astype(jnp.int32)
    out = _gather(table, idx_flat)
    return out.reshape(BATCH, HIST_LEN, EMBED_DIM)


# speedup vs baseline: 1.0114x; 1.0114x over previous
import functools

import jax
import jax.numpy as jnp
from jax import lax
from jax.experimental import pallas as pl
from jax.experimental.pallas import tpu as pltpu
from jax.experimental.pallas import tpu_sc as plsc

VOCAB = 100000
EMBED_DIM = 64
BATCH = 4096
HIST_LEN = 200

N = BATCH * HIST_LEN
NC = 2
NS = 16
NW = NC * NS
B_PER_W = N // NW
CHUNK = 128
NBUF = 8
N_CHUNKS = B_PER_W // CHUNK
N_STEPS = N_CHUNKS // NBUF


def _gather_body(table_hbm, idx_hbm, out_hbm, idx_v, rows_v, *sems):
    gsems = sems[:NBUF]
    ssems = sems[NBUF:]
    wid = lax.axis_index("s") * NC + lax.axis_index("c")
    base = wid * B_PER_W
    pltpu.sync_copy(idx_hbm.at[pl.ds(base, B_PER_W)], idx_v)

    def rows_ref(b):
        return rows_v.at[pl.ds(b * CHUNK, CHUNK)]

    def fire_gather(c, b):
        pltpu.async_copy(
            table_hbm.at[idx_v.at[pl.ds(c * CHUNK, CHUNK)]], rows_ref(b), gsems[b]
        )

    def wait_gather(b):
        pltpu.make_async_copy(
            table_hbm.at[idx_v.at[pl.ds(0, CHUNK)]], rows_ref(b), gsems[b]
        ).wait()

    def fire_store(c, b):
        pltpu.async_copy(
            rows_ref(b), out_hbm.at[pl.ds(base + c * CHUNK, CHUNK)], ssems[b]
        )

    def wait_store(b):
        pltpu.make_async_copy(
            rows_ref(b), out_hbm.at[pl.ds(base, CHUNK)], ssems[b]
        ).wait()

    for b in range(NBUF):
        fire_gather(b, b)

    @pl.loop(0, N_STEPS - 1)
    def _step(s):
        for b in range(NBUF):
            wait_gather(b)
            fire_store(s * NBUF + b, b)
        for b in range(NBUF):
            wait_store(b)
            fire_gather(s * NBUF + b + NBUF, b)

    for b in range(NBUF):
        c = (N_STEPS - 1) * NBUF + b
        wait_gather(b)
        fire_store(c, b)
    for b in range(NBUF):
        wait_store(b)


@jax.jit
def _gather(table, idx_flat):
    run = pl.kernel(
        _gather_body,
        out_type=jax.ShapeDtypeStruct((N, EMBED_DIM), jnp.float32),
        mesh=plsc.VectorSubcoreMesh(core_axis_name="c", subcore_axis_name="s"),
        scratch_types=[
            pltpu.VMEM((B_PER_W,), jnp.int32),
            pltpu.VMEM((NBUF * CHUNK, EMBED_DIM), jnp.float32),
        ]
        + [pltpu.SemaphoreType.DMA] * (2 * NBUF),
        compiler_params=pltpu.CompilerParams(use_tc_tiling_on_sc=False),
    )
    return run(table, idx_flat)


def kernel(table, index):
    idx_flat = index.reshape(-1).astype(jnp.int32)
    out = _gather(table, idx_flat)
    return out.reshape(BATCH, HIST_LEN, EMBED_DIM)


# speedup vs baseline: 1.0123x; 1.0009x over previous
import functools

import jax
import jax.numpy as jnp
from jax import lax
from jax.experimental import pallas as pl
from jax.experimental.pallas import tpu as pltpu
from jax.experimental.pallas import tpu_sc as plsc

VOCAB = 100000
EMBED_DIM = 64
BATCH = 4096
HIST_LEN = 200

N = BATCH * HIST_LEN
NC = 2
NS = 16
NW = NC * NS
B_PER_W = N // NW
CHUNK = 128
NBUF = 8
HALF = NBUF // 2
N_CHUNKS = B_PER_W // CHUNK
N_STEPS = N_CHUNKS // NBUF


def _gather_body(table_hbm, idx_hbm, out_hbm, idx_v, rows_v, *sems):
    gsems = sems[:NBUF]
    ssems = sems[NBUF:]
    wid = lax.axis_index("s") * NC + lax.axis_index("c")
    base = wid * B_PER_W
    pltpu.sync_copy(idx_hbm.at[pl.ds(base, B_PER_W)], idx_v)

    def rows_ref(b):
        return rows_v.at[pl.ds(b * CHUNK, CHUNK)]

    def fire_gather(c, b):
        pltpu.async_copy(
            table_hbm.at[idx_v.at[pl.ds(c * CHUNK, CHUNK)]], rows_ref(b), gsems[b]
        )

    def wait_gather(b):
        pltpu.make_async_copy(
            table_hbm.at[idx_v.at[pl.ds(0, CHUNK)]], rows_ref(b), gsems[b]
        ).wait()

    def fire_store(c, b):
        pltpu.async_copy(
            rows_ref(b), out_hbm.at[pl.ds(base + c * CHUNK, CHUNK)], ssems[b]
        )

    def wait_store(b):
        pltpu.make_async_copy(
            rows_ref(b), out_hbm.at[pl.ds(base, CHUNK)], ssems[b]
        ).wait()

    for b in range(HALF):
        fire_gather(b, b)

    for b in range(NBUF):
        wait_gather(b)
        fire_store(b, b)
        b2 = (b + HALF) % NBUF
        if b < HALF:
            fire_gather(b + HALF, b2)
        else:
            wait_store(b2)
            fire_gather(b + HALF, b2)

    @pl.loop(1, N_STEPS - 1)
    def _step(s):
        c0 = s * NBUF
        for b in range(NBUF):
            wait_gather(b)
            fire_store(c0 + b, b)
            b2 = (b + HALF) % NBUF
            wait_store(b2)
            fire_gather(c0 + b + HALF, b2)

    c0 = (N_STEPS - 1) * NBUF
    for b in range(NBUF):
        wait_gather(b)
        fire_store(c0 + b, b)
        b2 = (b + HALF) % NBUF
        wait_store(b2)
        if b < HALF:
            fire_gather(c0 + b + HALF, b2)
    for b in range(HALF, NBUF):
        wait_store(b)


@jax.jit
def _gather(table, idx_flat):
    run = pl.kernel(
        _gather_body,
        out_type=jax.ShapeDtypeStruct((N, EMBED_DIM), jnp.float32),
        mesh=plsc.VectorSubcoreMesh(core_axis_name="c", subcore_axis_name="s"),
        scratch_types=[
            pltpu.VMEM((B_PER_W,), jnp.int32),
            pltpu.VMEM((NBUF * CHUNK, EMBED_DIM), jnp.float32),
        ]
        + [pltpu.SemaphoreType.DMA] * (2 * NBUF),
        compiler_params=pltpu.CompilerParams(use_tc_tiling_on_sc=False),
    )
    return run(table, idx_flat)


def kernel(table, index):
    idx_flat = index.reshape(-1).astype(jnp.int32)
    out = _gather(table, idx_flat)
    return out.reshape(BATCH, HIST_LEN, EMBED_DIM)
